# baseline (speedup 1.0000x reference)
import functools

import jax
import jax.numpy as jnp
from jax import lax
from jax.experimental import pallas as pl
from jax.experimental.pallas import tpu as pltpu
from jax.experimental.pallas import tpu_sc as plsc

_H = 32
_W = 32
_D = 256
_B = 64

_mesh = plsc.VectorSubcoreMesh(core_axis_name="c", subcore_axis_name="s")


@functools.partial(
    pl.kernel,
    mesh=_mesh,
    out_type=[
        jax.ShapeDtypeStruct((_B, 2 * _D), jnp.float32),
        jax.ShapeDtypeStruct((_B, _H * _W, 2 * _D), jnp.float32),
    ],
    scratch_types=[
        pltpu.VMEM((_W, 2 * _D), jnp.float32),
        pltpu.VMEM((8, 2 * _D), jnp.float32),
        pltpu.SemaphoreType.DMA,
        pltpu.SemaphoreType.DMA,
    ],
)
def _sc_emb(row_hbm, col_hbm, pose_hbm, p_out, m_out, patbuf, pbuf, sem_in, sem_out):
    nc = 2
    wid = lax.axis_index("s") * nc + lax.axis_index("c")
    yp1 = wid + 1

    fills = []
    for xi in range(_W):
        fills.append(
            pltpu.make_async_copy(
                col_hbm.at[xi + 1], patbuf.at[xi, pl.ds(0, _D)], sem_in
            )
        )
        fills.append(
            pltpu.make_async_copy(
                row_hbm.at[yp1], patbuf.at[xi, pl.ds(_D, _D)], sem_in
            )
        )
    for f in fills:
        f.start()
    for f in fills:
        f.wait()

    outs = [
        pltpu.make_async_copy(
            patbuf, m_out.at[b, pl.ds(wid * _W, _W)], sem_out
        )
        for b in range(_B)
    ]
    for o in outs:
        o.start()

    @pl.when(wid == 0)
    def _p():
        pf = []
        for r in range(8):
            pf.append(
                pltpu.make_async_copy(pose_hbm.at[0], pbuf.at[r, pl.ds(0, _D)], sem_in)
            )
            pf.append(
                pltpu.make_async_copy(pose_hbm.at[0], pbuf.at[r, pl.ds(_D, _D)], sem_in)
            )
        for f in pf:
            f.start()
        for f in pf:
            f.wait()
        po = [
            pltpu.make_async_copy(pbuf, p_out.at[pl.ds(8 * k, 8)], sem_in)
            for k in range(_B // 8)
        ]
        for o in po:
            o.start()
        for o in po:
            o.wait()

    for o in outs:
        o.wait()


def kernel(x, row_embed, col_embed, pose_token_embed):
    B = x.shape[0]
    h, w = x.shape[-2], x.shape[-1]
    d = col_embed.shape[1]
    p_emb, m_bhwc = _sc_emb(row_embed, col_embed, pose_token_embed)
    m_emb = m_bhwc.reshape(B, h, w, 2 * d).transpose(0, 3, 1, 2)
    return (p_emb, m_emb)
